# baseline (speedup 1.0000x reference)
import functools

import jax
import jax.numpy as jnp
from jax import lax
from jax.experimental import pallas as pl
from jax.experimental.pallas import tpu as pltpu
from jax.experimental.pallas import tpu_sc as plsc

D_MODEL = 64
SEQ = 50
BATCH = 4096
ROWS = BATCH * SEQ
NUM_CORES = 2
NUM_SUBCORES = 16
NW = NUM_CORES * NUM_SUBCORES
PER_W = ROWS // NW
CHUNK = 800
NCHUNK = PER_W // CHUNK
LANES = 16
REPS = CHUNK // SEQ
DPARTS = D_MODEL // LANES

_mesh = plsc.VectorSubcoreMesh(core_axis_name="c", subcore_axis_name="s")


@functools.partial(
    pl.kernel,
    out_type=jax.ShapeDtypeStruct((BATCH, SEQ, D_MODEL), jnp.float32),
    mesh=_mesh,
    scratch_types=[
        pltpu.VMEM((CHUNK,), jnp.int32),
        pltpu.VMEM((CHUNK, D_MODEL), jnp.float32),
        pltpu.VMEM((SEQ * D_MODEL,), jnp.float32),
        pltpu.SemaphoreType.DMA,
    ],
    compiler_params=pltpu.CompilerParams(use_tc_tiling_on_sc=False),
)
def _stay_embedding(codes_hbm, table_hbm, pe_hbm, out_hbm, idx_v, buf, pe_v, sem):
    wid = lax.axis_index("s") * NUM_CORES + lax.axis_index("c")
    base = wid * PER_W
    pltpu.sync_copy(pe_hbm, pe_v)

    def chunk_body(ci, carry):
        cb = base + ci * CHUNK
        pltpu.sync_copy(codes_hbm.at[pl.ds(cb, CHUNK)], idx_v)
        pltpu.async_copy(table_hbm.at[idx_v], buf, sem).wait()

        def pe_body(t, c2):
            for dp in range(DPARTS):
                pe_vec = pe_v[pl.ds(t * D_MODEL + dp * LANES, LANES)]
                for r in range(REPS):
                    plsc.addupdate(
                        buf.at[t + r * SEQ, pl.ds(dp * LANES, LANES)], pe_vec
                    )
            return c2

        lax.fori_loop(0, SEQ, pe_body, 0)
        b0 = cb // SEQ
        for bb in range(REPS):
            pltpu.sync_copy(buf.at[pl.ds(bb * SEQ, SEQ)], out_hbm.at[b0 + bb])
        return carry

    lax.fori_loop(0, NCHUNK, chunk_body, 0)


def kernel(codes, table, pe):
    codes_flat = codes.reshape(ROWS)
    pe_flat = pe[:SEQ].reshape(SEQ * D_MODEL)
    return _stay_embedding(codes_flat, table, pe_flat)


# speedup vs baseline: 1.4404x; 1.4404x over previous
import functools

import jax
import jax.numpy as jnp
from jax import lax
from jax.experimental import pallas as pl
from jax.experimental.pallas import tpu as pltpu
from jax.experimental.pallas import tpu_sc as plsc

D_MODEL = 64
SEQ = 50
BATCH = 4096
NUM_CORES = 2
NUM_SUBCORES = 16
NW = NUM_CORES * NUM_SUBCORES
B_PER_W = BATCH // NW
CPB = 8
NCHUNK = B_PER_W // CPB
CROWS = CPB * SEQ
LANES = 16
NVEC = CROWS // LANES
DPARTS = D_MODEL // LANES

_mesh = plsc.VectorSubcoreMesh(core_axis_name="c", subcore_axis_name="s")


@functools.partial(
    pl.kernel,
    out_type=jax.ShapeDtypeStruct((BATCH, SEQ, D_MODEL), jnp.float32),
    mesh=_mesh,
    scratch_types=[
        pltpu.VMEM((CROWS,), jnp.int32),
        pltpu.VMEM((CROWS, D_MODEL), jnp.float32),
        pltpu.VMEM((SEQ * D_MODEL,), jnp.float32),
        pltpu.SemaphoreType.DMA,
    ],
)
def _stay_embedding(codes_hbm, table_hbm, pe_hbm, out_hbm, idx_v, buf, pe_v, sem):
    wid = lax.axis_index("s") * NUM_CORES + lax.axis_index("c")
    pltpu.sync_copy(pe_hbm, pe_v)

    def chunk_body(ci, carry):
        b0 = wid * B_PER_W + ci * CPB
        pltpu.sync_copy(codes_hbm.at[pl.ds(b0 * SEQ, CROWS)], idx_v)

        def fire_group(g, c2):
            vec = idx_v[pl.ds(g * LANES, LANES)]
            slot = g * LANES
            for j in range(LANES):
                code = vec[j]
                pltpu.make_async_copy(
                    table_hbm.at[code], buf.at[slot + j], sem
                ).start()
            return c2

        lax.fori_loop(0, NVEC, fire_group, 0)

        def drain_row(j, c2):
            pltpu.make_async_copy(table_hbm.at[0], buf.at[0], sem).wait()
            return c2

        lax.fori_loop(0, CROWS, drain_row, 0)

        def pe_body(t, c2):
            for dp in range(DPARTS):
                pe_vec = pe_v[pl.ds(t * D_MODEL + dp * LANES, LANES)]
                for bi in range(CPB):
                    plsc.addupdate(
                        buf.at[bi * SEQ + t, pl.ds(dp * LANES, LANES)], pe_vec
                    )
            return c2

        lax.fori_loop(0, SEQ, pe_body, 0)
        for bb in range(CPB):
            pltpu.sync_copy(buf.at[pl.ds(bb * SEQ, SEQ)], out_hbm.at[b0 + bb])
        return carry

    lax.fori_loop(0, NCHUNK, chunk_body, 0)


def kernel(codes, table, pe):
    codes_flat = codes.reshape(BATCH * SEQ)
    pe_flat = pe[:SEQ].reshape(SEQ * D_MODEL)
    return _stay_embedding(codes_flat, table, pe_flat)
